# baseline (speedup 1.0000x reference)
import jax
import jax.numpy as jnp
from jax import lax
from jax.experimental import pallas as pl
from jax.experimental.pallas import tpu as pltpu
from jax.experimental.pallas import tpu_sc as plsc

_D = 128
_ROWS = 16


def _body(i1_hbm, i2_hbm, i3_hbm, tab_hbm, out_hbm,
          i1_v, i2_v, i3_v, idx_v, rows_v, sem):
    c = lax.axis_index("c")
    s = lax.axis_index("s")

    @pl.when(jnp.logical_and(c == 0, s == 0))
    def _():
        pltpu.sync_copy(i1_hbm, i1_v)
        pltpu.sync_copy(i2_hbm, i2_v)
        pltpu.sync_copy(i3_hbm, i3_v)
        idx_v[...] = i1_v[...] * 4096 + i2_v[...] * 64 + i3_v[...]
        pltpu.async_copy(tab_hbm.at[idx_v], rows_v, sem).wait()
        pltpu.sync_copy(rows_v, out_hbm)


def _gather16(i1, i2, i3, tab):
    mesh = plsc.VectorSubcoreMesh(core_axis_name="c", subcore_axis_name="s")
    f = pl.kernel(
        _body,
        mesh=mesh,
        out_type=jax.ShapeDtypeStruct((_ROWS, _D), jnp.float32),
        scratch_types=[
            pltpu.VMEM((16,), jnp.int32),
            pltpu.VMEM((16,), jnp.int32),
            pltpu.VMEM((16,), jnp.int32),
            pltpu.VMEM((16,), jnp.int32),
            pltpu.VMEM((_ROWS, _D), jnp.float32),
            pltpu.SemaphoreType.DMA,
        ],
    )
    return f(i1, i2, i3, tab)


def kernel(x, index1, index2, index3):
    tab = x.reshape(-1, _D)
    i1 = jnp.pad(jnp.broadcast_to(index1.astype(jnp.int32), (4, 3)).reshape(-1), (0, 4))
    i2 = jnp.pad(jnp.broadcast_to(index2.astype(jnp.int32), (4, 3)).reshape(-1), (0, 4))
    i3 = jnp.pad(index3.reshape(-1).astype(jnp.int32), (0, 4))
    out = _gather16(i1, i2, i3, tab)
    return out[:12].reshape(4, 3, _D)


# speedup vs baseline: 1.1590x; 1.1590x over previous
import jax
import jax.numpy as jnp
from jax import lax
from jax.experimental import pallas as pl
from jax.experimental.pallas import tpu as pltpu
from jax.experimental.pallas import tpu_sc as plsc

_D = 128
_OUT = 12


def _body(pack_hbm, tab_hbm, out_hbm, pack_v, idx_v, rows_v, sem):
    @pl.when(jnp.logical_and(lax.axis_index("c") == 0, lax.axis_index("s") == 0))
    def _():
        pltpu.sync_copy(pack_hbm, pack_v)
        v1 = pack_v[pl.ds(0, 16)]
        v2 = pack_v[pl.ds(16, 16)]
        v3 = pack_v[pl.ds(32, 16)]
        idx_v[...] = v1 * 4096 + v2 * 64 + v3
        pltpu.async_copy(tab_hbm.at[idx_v], rows_v, sem).wait()
        pltpu.sync_copy(rows_v.at[pl.ds(0, _OUT)], out_hbm)


def _gather12(pack, tab):
    mesh = plsc.VectorSubcoreMesh(core_axis_name="c", subcore_axis_name="s",
                                  num_cores=1)
    f = pl.kernel(
        _body,
        mesh=mesh,
        out_type=jax.ShapeDtypeStruct((_OUT, _D), jnp.float32),
        scratch_types=[
            pltpu.VMEM((48,), jnp.int32),
            pltpu.VMEM((16,), jnp.int32),
            pltpu.VMEM((16, _D), jnp.float32),
            pltpu.SemaphoreType.DMA,
        ],
    )
    return f(pack, tab)


def kernel(x, index1, index2, index3):
    tab = x.reshape(-1, _D)
    i1 = jnp.pad(jnp.broadcast_to(index1.astype(jnp.int32), (4, 3)).reshape(-1), (0, 4))
    i2 = jnp.pad(jnp.broadcast_to(index2.astype(jnp.int32), (4, 3)).reshape(-1), (0, 4))
    i3 = jnp.pad(index3.reshape(-1).astype(jnp.int32), (0, 4))
    pack = jnp.concatenate([i1, i2, i3])
    out = _gather12(pack, tab)
    return out.reshape(4, 3, _D)
